# Initial kernel scaffold; baseline (speedup 1.0000x reference)
import jax
import jax.numpy as jnp
from jax.experimental import pallas as pl


def kernel(embeddings, batch, W1, b1, W2, b2):
    raise NotImplementedError("write your pallas kernel here")


# speedup vs baseline: 4.0062x; 4.0062x over previous
import jax
import jax.numpy as jnp
from jax import lax
from jax.experimental import pallas as pl
from jax.experimental.pallas import tpu as pltpu
from jax.experimental.pallas import tpu_sc as plsc

N = 100000
D = 128
G = 1024

CH = 80
NCH = N // CH
NC = 2
NS = 16
NW = NC * NS
CPW = -(-NCH // NW)
CW = 128
GPT = G // NS


def _segsum_body(emb, idx_hbm, zsum, zcnt, ones_hbm,
                 sums_out, cnts_out,
                 rows_v, idx_v, ones_v, acc, cacc):
    c = lax.axis_index("c")
    s = lax.axis_index("s")
    wid = s * NC + c

    pltpu.sync_copy(zsum.at[pl.ds(s * GPT, GPT)], acc.at[pl.ds(s * GPT, GPT)])
    pltpu.sync_copy(zcnt.at[pl.ds(s * GPT, GPT)], cacc.at[pl.ds(s * GPT, GPT)])
    pltpu.sync_copy(ones_hbm, ones_v)
    plsc.subcore_barrier()

    def chunk(j, carry):
        k = wid + NW * j

        @pl.when(k < NCH)
        def _():
            r0 = k * CH
            pltpu.sync_copy(emb.at[pl.ds(r0, CH)], rows_v)
            pltpu.sync_copy(idx_hbm.at[pl.ds(r0, CH)], idx_v)
            pltpu.sync_copy(rows_v, acc.at[idx_v], add=True)
            pltpu.sync_copy(ones_v, cacc.at[idx_v], add=True)

        return carry

    lax.fori_loop(0, CPW, chunk, 0)
    plsc.subcore_barrier()

    pltpu.sync_copy(acc.at[pl.ds(s * GPT, GPT)],
                    sums_out.at[c, pl.ds(s * GPT, GPT)])
    pltpu.sync_copy(cacc.at[pl.ds(s * GPT, GPT)],
                    cnts_out.at[c, pl.ds(s * GPT, GPT)])


_segsum = pl.kernel(
    _segsum_body,
    out_type=(
        jax.ShapeDtypeStruct((NC, G, D), jnp.float32),
        jax.ShapeDtypeStruct((NC, G, CW), jnp.float32),
    ),
    mesh=plsc.VectorSubcoreMesh(core_axis_name="c", subcore_axis_name="s"),
    scratch_types=[
        pltpu.VMEM((CH, D), jnp.float32),
        pltpu.VMEM((CH,), jnp.int32),
        pltpu.VMEM((CH, CW), jnp.float32),
        pltpu.VMEM_SHARED((G, D), jnp.float32),
        pltpu.VMEM_SHARED((G, CW), jnp.float32),
    ],
)


def _ffn_body(sums_ref, cnts_ref, w1_ref, b1_ref, w2_ref, b2_ref, out_ref):
    sums = sums_ref[0] + sums_ref[1]
    cnt = cnts_ref[0, :, 0] + cnts_ref[1, :, 0]
    pool = sums / jnp.maximum(cnt, 1.0)[:, None]
    x = jnp.dot(pool, w1_ref[...], preferred_element_type=jnp.float32)
    x = jnp.maximum(x + b1_ref[...], 0.0)
    out = jnp.dot(x, w2_ref[...], preferred_element_type=jnp.float32)
    out_ref[...] = out + b2_ref[0, 0]


def _ffn(sums, cnts, W1, b1, W2, b2):
    return pl.pallas_call(
        _ffn_body,
        out_shape=jax.ShapeDtypeStruct((G, 1), jnp.float32),
    )(sums, cnts, W1, b1, W2, b2)


def kernel(embeddings, batch, W1, b1, W2, b2):
    idx = batch.astype(jnp.int32)
    zsum = jnp.zeros((G, D), jnp.float32)
    zcnt = jnp.zeros((G, CW), jnp.float32)
    ones = jnp.ones((CH, CW), jnp.float32)
    sums, cnts = _segsum(embeddings, idx, zsum, zcnt, ones)
    out = _ffn(sums, cnts, W1, b1.reshape(1, D), W2, b2.reshape(1, 1))
    return out[:, 0]


# speedup vs baseline: 6.2446x; 1.5588x over previous
import jax
import jax.numpy as jnp
from jax import lax
from jax.experimental import pallas as pl
from jax.experimental.pallas import tpu as pltpu
from jax.experimental.pallas import tpu_sc as plsc

N = 100000
D = 128
G = 1024

CH = 80
NCH = N // CH
NC = 2
NS = 16
NW = NC * NS
CPW = -(-NCH // NW)
CW = 128
GPT = G // NS


NJ = NCH // NW


def _segsum_body(emb, idx_hbm, zsum, zcnt, ones_hbm,
                 sums_out, cnts_out,
                 rows0, rows1, idx0, idx1, ones_v, acc, cacc,
                 sr0, sr1, si0, si1, ss0, ss1, sc0, sc1):
    c = lax.axis_index("c")
    s = lax.axis_index("s")
    wid = s * NC + c

    pltpu.sync_copy(zsum.at[pl.ds(s * GPT, GPT)], acc.at[pl.ds(s * GPT, GPT)])
    pltpu.sync_copy(zcnt.at[pl.ds(s * GPT, GPT)], cacc.at[pl.ds(s * GPT, GPT)])
    pltpu.sync_copy(ones_hbm, ones_v)
    plsc.subcore_barrier()

    rows = (rows0, rows1)
    idxb = (idx0, idx1)
    srs = (sr0, sr1)
    sis = (si0, si1)
    sss = (ss0, ss1)
    scs = (sc0, sc1)

    def start_in(j, b):
        r0 = (wid + NW * j) * CH
        pltpu.async_copy(emb.at[pl.ds(r0, CH)], rows[b], srs[b])
        pltpu.async_copy(idx_hbm.at[pl.ds(r0, CH)], idxb[b], sis[b])

    def wait_in(j, b):
        r0 = (wid + NW * j) * CH
        pltpu.make_async_copy(emb.at[pl.ds(r0, CH)], rows[b], srs[b]).wait()
        pltpu.make_async_copy(idx_hbm.at[pl.ds(r0, CH)], idxb[b], sis[b]).wait()

    def phase(j, b, more):
        wait_in(j, b)
        d1 = pltpu.async_copy(rows[b], acc.at[idxb[b]], sss[b], add=True)
        d2 = pltpu.async_copy(ones_v, cacc.at[idxb[b]], scs[b], add=True)

        @pl.when(more)
        def _():
            start_in(j + 1, 1 - b)

        d1.wait()
        d2.wait()

    start_in(0, 0)

    def pair(p, carry):
        j0 = 2 * p
        phase(j0, 0, j0 + 1 <= NJ - 1)

        @pl.when(j0 + 1 <= NJ - 1)
        def _():
            phase(j0 + 1, 1, j0 + 2 <= NJ - 1)

        return carry

    lax.fori_loop(0, (NJ + 1) // 2, pair, 0)

    @pl.when(wid + NW * NJ < NCH)
    def _():
        r0 = (wid + NW * NJ) * CH
        pltpu.sync_copy(emb.at[pl.ds(r0, CH)], rows1)
        pltpu.sync_copy(idx_hbm.at[pl.ds(r0, CH)], idx1)
        pltpu.sync_copy(rows1, acc.at[idx1], add=True)
        pltpu.sync_copy(ones_v, cacc.at[idx1], add=True)

    plsc.subcore_barrier()

    pltpu.sync_copy(acc.at[pl.ds(s * GPT, GPT)],
                    sums_out.at[c, pl.ds(s * GPT, GPT)])
    pltpu.sync_copy(cacc.at[pl.ds(s * GPT, GPT)],
                    cnts_out.at[c, pl.ds(s * GPT, GPT)])


_segsum = pl.kernel(
    _segsum_body,
    out_type=(
        jax.ShapeDtypeStruct((NC, G, D), jnp.float32),
        jax.ShapeDtypeStruct((NC, G, CW), jnp.float32),
    ),
    mesh=plsc.VectorSubcoreMesh(core_axis_name="c", subcore_axis_name="s"),
    scratch_types=[
        pltpu.VMEM((CH, D), jnp.float32),
        pltpu.VMEM((CH, D), jnp.float32),
        pltpu.VMEM((CH,), jnp.int32),
        pltpu.VMEM((CH,), jnp.int32),
        pltpu.VMEM((CH, CW), jnp.float32),
        pltpu.VMEM_SHARED((G, D), jnp.float32),
        pltpu.VMEM_SHARED((G, CW), jnp.float32),
        pltpu.SemaphoreType.DMA,
        pltpu.SemaphoreType.DMA,
        pltpu.SemaphoreType.DMA,
        pltpu.SemaphoreType.DMA,
        pltpu.SemaphoreType.DMA,
        pltpu.SemaphoreType.DMA,
        pltpu.SemaphoreType.DMA,
        pltpu.SemaphoreType.DMA,
    ],
)


def _ffn_body(sums_ref, cnts_ref, w1_ref, b1_ref, w2_ref, b2_ref, out_ref):
    sums = sums_ref[0] + sums_ref[1]
    cnt = cnts_ref[0, :, 0] + cnts_ref[1, :, 0]
    pool = sums / jnp.maximum(cnt, 1.0)[:, None]
    x = jnp.dot(pool, w1_ref[...], preferred_element_type=jnp.float32)
    x = jnp.maximum(x + b1_ref[...], 0.0)
    out = jnp.dot(x, w2_ref[...], preferred_element_type=jnp.float32)
    out_ref[...] = out + b2_ref[0, 0]


def _ffn(sums, cnts, W1, b1, W2, b2):
    return pl.pallas_call(
        _ffn_body,
        out_shape=jax.ShapeDtypeStruct((G, 1), jnp.float32),
    )(sums, cnts, W1, b1, W2, b2)


def kernel(embeddings, batch, W1, b1, W2, b2):
    idx = batch.astype(jnp.int32)
    zsum = jnp.zeros((G, D), jnp.float32)
    zcnt = jnp.zeros((G, CW), jnp.float32)
    ones = jnp.ones((CH, CW), jnp.float32)
    sums, cnts = _segsum(embeddings, idx, zsum, zcnt, ones)
    out = _ffn(sums, cnts, W1, b1.reshape(1, D), W2, b2.reshape(1, 1))
    return out[:, 0]
